# baseline (speedup 1.0000x reference)
import functools

import jax
import jax.numpy as jnp
from jax import lax
from jax.experimental import pallas as pl
from jax.experimental.pallas import tpu as pltpu
from jax.experimental.pallas import tpu_sc as plsc

_B = 1024
_D = 64
_V = 100000
_V_TILE = 512


def _sc_gather(embeddings, target):
    info = plsc.get_sparse_core_info()
    nw = info.num_cores * info.num_subcores
    b_per_w = _B // nw

    mesh = plsc.VectorSubcoreMesh(core_axis_name="c", subcore_axis_name="s")

    @functools.partial(
        pl.kernel,
        mesh=mesh,
        out_type=jax.ShapeDtypeStruct((_B, _D), jnp.float32),
        scratch_types=[
            pltpu.VMEM((b_per_w,), jnp.int32),
            pltpu.VMEM((b_per_w, _D), jnp.float32),
            pltpu.SemaphoreType.DMA,
        ],
        compiler_params=pltpu.CompilerParams(use_tc_tiling_on_sc=False),
    )
    def gather_k(table_hbm, idx_hbm, out_hbm, idx_v, rows_v, sem):
        wid = lax.axis_index("s") * info.num_cores + lax.axis_index("c")
        base = wid * b_per_w
        pltpu.sync_copy(idx_hbm.at[pl.ds(base, b_per_w)], idx_v)
        pltpu.async_copy(table_hbm.at[idx_v], rows_v, sem).wait()
        pltpu.sync_copy(rows_v, out_hbm.at[pl.ds(base, b_per_w)])

    return gather_k(embeddings, target)


def _proj_body(embed_ref, w_ref, b_ref, out_ref):
    acc = lax.dot_general(
        embed_ref[...],
        w_ref[...],
        (((1,), (1,)), ((), ())),
        preferred_element_type=jnp.float32,
    )
    out_ref[...] = acc + b_ref[...][None, :]


def _tc_project(embed, W, b):
    grid = (pl.cdiv(_V, _V_TILE),)
    return pl.pallas_call(
        _proj_body,
        grid=grid,
        in_specs=[
            pl.BlockSpec((_B, _D), lambda j: (0, 0)),
            pl.BlockSpec((_V_TILE, _D), lambda j: (j, 0)),
            pl.BlockSpec((_V_TILE,), lambda j: (j,)),
        ],
        out_specs=pl.BlockSpec((_B, _V_TILE), lambda j: (0, j)),
        out_shape=jax.ShapeDtypeStruct((_B, _V), jnp.float32),
    )(embed, W, b)


def kernel(target, embeddings, W, b):
    embed = _sc_gather(embeddings, target.astype(jnp.int32))
    return _tc_project(embed, W, b)


# speedup vs baseline: 1.1370x; 1.1370x over previous
import functools

import jax
import jax.numpy as jnp
from jax import lax
from jax.experimental import pallas as pl
from jax.experimental.pallas import tpu as pltpu
from jax.experimental.pallas import tpu_sc as plsc

_B = 1024
_D = 64
_V = 100000
_VT = 2048
_NSTEPS = 49
_TAIL = _V - (_NSTEPS - 1) * _VT
_NBUF = 4
_LAST_SLOT = (_NSTEPS - 1) % _NBUF


def _sc_gather(embeddings, target):
    info = plsc.get_sparse_core_info()
    nw = info.num_cores * info.num_subcores
    b_per_w = _B // nw

    mesh = plsc.VectorSubcoreMesh(core_axis_name="c", subcore_axis_name="s")

    @functools.partial(
        pl.kernel,
        mesh=mesh,
        out_type=jax.ShapeDtypeStruct((_B, _D), jnp.float32),
        scratch_types=[
            pltpu.VMEM((b_per_w,), jnp.int32),
            pltpu.VMEM((b_per_w, _D), jnp.float32),
            pltpu.SemaphoreType.DMA,
        ],
        compiler_params=pltpu.CompilerParams(use_tc_tiling_on_sc=False),
    )
    def gather_k(table_hbm, idx_hbm, out_hbm, idx_v, rows_v, sem):
        wid = lax.axis_index("s") * info.num_cores + lax.axis_index("c")
        base = wid * b_per_w
        pltpu.sync_copy(idx_hbm.at[pl.ds(base, b_per_w)], idx_v)
        pltpu.async_copy(table_hbm.at[idx_v], rows_v, sem).wait()
        pltpu.sync_copy(rows_v, out_hbm.at[pl.ds(base, b_per_w)])

    return gather_k(embeddings, target)


def _proj_body(embed_ref, w_ref, b_ref, out_hbm, acc_ref, tail_ref, sems):
    j = pl.program_id(0)
    slot = lax.rem(j, _NBUF)

    @pl.when(j >= _NBUF)
    def _():
        pltpu.make_async_copy(
            acc_ref.at[slot],
            out_hbm.at[:, pl.ds((j - _NBUF) * _VT, _VT)],
            sems.at[slot],
        ).wait()

    acc = lax.dot_general(
        embed_ref[...],
        w_ref[...],
        (((1,), (1,)), ((), ())),
        preferred_element_type=jnp.float32,
    ) + b_ref[...][None, :]

    @pl.when(j < _NSTEPS - 1)
    def _():
        acc_ref[slot] = acc
        pltpu.make_async_copy(
            acc_ref.at[slot],
            out_hbm.at[:, pl.ds(j * _VT, _VT)],
            sems.at[slot],
        ).start()

    @pl.when(j == _NSTEPS - 1)
    def _():
        tail_ref[...] = acc[:, : _TAIL]
        pltpu.make_async_copy(
            tail_ref,
            out_hbm.at[:, pl.ds((_NSTEPS - 1) * _VT, _TAIL)],
            sems.at[_LAST_SLOT],
        ).start()
        for k in range(1, _NBUF):
            jj = _NSTEPS - 1 - _NBUF + k
            pltpu.make_async_copy(
                acc_ref.at[jj % _NBUF],
                out_hbm.at[:, pl.ds(jj * _VT, _VT)],
                sems.at[jj % _NBUF],
            ).wait()
        pltpu.make_async_copy(
            tail_ref,
            out_hbm.at[:, pl.ds((_NSTEPS - 1) * _VT, _TAIL)],
            sems.at[_LAST_SLOT],
        ).wait()


def _tc_project(embed, W, b):
    return pl.pallas_call(
        _proj_body,
        grid=(_NSTEPS,),
        in_specs=[
            pl.BlockSpec((_B, _D), lambda j: (0, 0)),
            pl.BlockSpec((_VT, _D), lambda j: (j, 0)),
            pl.BlockSpec((_VT,), lambda j: (j,)),
        ],
        out_specs=pl.BlockSpec(memory_space=pl.ANY),
        out_shape=jax.ShapeDtypeStruct((_B, _V), jnp.float32),
        scratch_shapes=[
            pltpu.VMEM((_NBUF, _B, _VT), jnp.float32),
            pltpu.VMEM((_B, _TAIL), jnp.float32),
            pltpu.SemaphoreType.DMA((_NBUF,)),
        ],
    )(embed, W, b)


def kernel(target, embeddings, W, b):
    embed = _sc_gather(embeddings, target.astype(jnp.int32))
    return _tc_project(embed, W, b)


# speedup vs baseline: 3.1785x; 2.7955x over previous
import functools

import jax
import jax.numpy as jnp
from jax import lax
from jax.experimental import pallas as pl
from jax.experimental.pallas import tpu as pltpu
from jax.experimental.pallas import tpu_sc as plsc

_B = 1024
_D = 64
_V = 100000
_VT = 2048


def _sc_gather(table_pairs, target):
    info = plsc.get_sparse_core_info()
    nw = info.num_cores * info.num_subcores
    bw = _B // nw

    mesh = plsc.VectorSubcoreMesh(core_axis_name="c", subcore_axis_name="s")

    @functools.partial(
        pl.kernel,
        mesh=mesh,
        out_type=jax.ShapeDtypeStruct((_B, _D), jnp.float32),
        scratch_types=[
            pltpu.VMEM((bw,), jnp.int32),
            pltpu.VMEM((bw,), jnp.int32),
            pltpu.VMEM((bw, 2 * _D), jnp.float32),
            pltpu.VMEM((bw, _D), jnp.float32),
            pltpu.SemaphoreType.DMA,
        ],
        compiler_params=pltpu.CompilerParams(needs_layout_passes=False),
    )
    def gather_k(table_hbm, idx_hbm, out_hbm, idx_v, sup_v, pair_v, row_v,
                 sem):
        wid = lax.axis_index("s") * info.num_cores + lax.axis_index("c")
        base = wid * bw
        pltpu.sync_copy(idx_hbm.at[pl.ds(base, bw)], idx_v)
        for c in range(bw // 16):
            sl = pl.ds(c * 16, 16)
            sup_v[sl] = lax.shift_right_logical(idx_v[sl], 1)
        pltpu.async_copy(table_hbm.at[sup_v], pair_v, sem).wait()
        iota16 = lax.iota(jnp.int32, 16)
        for g in range(bw // 16):
            rows_g = iota16 + g * 16
            off_g = (idx_v[pl.ds(g * 16, 16)] & 1) * _D
            for d in range(_D):
                vals = plsc.load_gather(pair_v, [rows_g, off_g + d])
                plsc.store_scatter(
                    row_v, [rows_g, jnp.full((16,), d, jnp.int32)], vals
                )
        pltpu.sync_copy(row_v, out_hbm.at[pl.ds(base, bw)])

    return gather_k(table_pairs, target)


def _proj_t_body(wt_ref, embed_ref, b_ref, out_ref):
    acc = lax.dot_general(
        wt_ref[...],
        embed_ref[...],
        (((0,), (1,)), ((), ())),
        preferred_element_type=jnp.float32,
    )
    ones = jnp.ones((1, _B), jnp.float32)
    bias = lax.dot_general(
        b_ref[...][None, :],
        ones,
        (((0,), (0,)), ((), ())),
        preferred_element_type=jnp.float32,
    )
    out_ref[...] = acc + bias


def _tc_project_t(Wt, embed, b):
    return pl.pallas_call(
        _proj_t_body,
        grid=(pl.cdiv(_V, _VT),),
        in_specs=[
            pl.BlockSpec((_D, _VT), lambda j: (0, j)),
            pl.BlockSpec((_B, _D), lambda j: (0, 0)),
            pl.BlockSpec((_VT,), lambda j: (j,)),
        ],
        out_specs=pl.BlockSpec((_VT, _B), lambda j: (j, 0)),
        out_shape=jax.ShapeDtypeStruct((_V, _B), jnp.float32),
    )(Wt, embed, b)


def kernel(target, embeddings, W, b):
    embed = _sc_gather(
        embeddings.reshape(_V // 2, 2 * _D), target.astype(jnp.int32)
    )
    out_t = _tc_project_t(W.T, embed, b)
    return out_t.T


# speedup vs baseline: 3.3375x; 1.0500x over previous
import functools

import jax
import jax.numpy as jnp
from jax import lax
from jax.experimental import pallas as pl
from jax.experimental.pallas import tpu as pltpu
from jax.experimental.pallas import tpu_sc as plsc

_B = 1024
_D = 64
_V = 100000
_VT = 2048


def _sc_gather(table_pad, target):
    info = plsc.get_sparse_core_info()
    nw = info.num_cores * info.num_subcores
    bw = _B // nw

    mesh = plsc.VectorSubcoreMesh(core_axis_name="c", subcore_axis_name="s")

    @functools.partial(
        pl.kernel,
        mesh=mesh,
        out_type=jax.ShapeDtypeStruct((_B, 2 * _D), jnp.float32),
        scratch_types=[
            pltpu.VMEM((bw,), jnp.int32),
            pltpu.VMEM((bw, 2 * _D), jnp.float32),
            pltpu.SemaphoreType.DMA,
        ],
        compiler_params=pltpu.CompilerParams(needs_layout_passes=False),
    )
    def gather_k(table_hbm, idx_hbm, out_hbm, idx_v, rows_v, sem):
        wid = lax.axis_index("s") * info.num_cores + lax.axis_index("c")
        base = wid * bw
        pltpu.sync_copy(idx_hbm.at[pl.ds(base, bw)], idx_v)
        pltpu.async_copy(table_hbm.at[idx_v], rows_v, sem).wait()
        pltpu.sync_copy(rows_v, out_hbm.at[pl.ds(base, bw)])

    return gather_k(table_pad, target)


def _proj_t_body(wt_ref, embed_ref, b_ref, out_ref):
    acc = lax.dot_general(
        wt_ref[...],
        embed_ref[..., : _D],
        (((0,), (1,)), ((), ())),
        preferred_element_type=jnp.float32,
    )
    ones = jnp.ones((1, _B), jnp.float32)
    bias = lax.dot_general(
        b_ref[...][None, :],
        ones,
        (((0,), (0,)), ((), ())),
        preferred_element_type=jnp.float32,
    )
    out_ref[...] = acc + bias


def _tc_project_t(Wt, embed_pad, b):
    return pl.pallas_call(
        _proj_t_body,
        grid=(pl.cdiv(_V, _VT),),
        in_specs=[
            pl.BlockSpec((_D, _VT), lambda j: (0, j)),
            pl.BlockSpec((_B, 2 * _D), lambda j: (0, 0)),
            pl.BlockSpec((_VT,), lambda j: (j,)),
        ],
        out_specs=pl.BlockSpec((_VT, _B), lambda j: (j, 0)),
        out_shape=jax.ShapeDtypeStruct((_V, _B), jnp.float32),
    )(Wt, embed_pad, b)


def kernel(target, embeddings, W, b):
    table_pad = jnp.pad(embeddings, ((0, 0), (0, _D)))
    embed_pad = _sc_gather(table_pad, target.astype(jnp.int32))
    out_t = _tc_project_t(W.T, embed_pad, b)
    return out_t.T


# speedup vs baseline: 3.9097x; 1.1714x over previous
import functools

import jax
import jax.numpy as jnp
from jax import lax
from jax.experimental import pallas as pl
from jax.experimental.pallas import tpu as pltpu
from jax.experimental.pallas import tpu_sc as plsc

_B = 1024
_D = 64
_V = 100000
_VT = 2048


def _sc_gather(table_pad, target):
    info = plsc.get_sparse_core_info()
    nw = info.num_cores * info.num_subcores
    bw = _B // nw

    mesh = plsc.VectorSubcoreMesh(core_axis_name="c", subcore_axis_name="s")

    @functools.partial(
        pl.kernel,
        mesh=mesh,
        out_type=jax.ShapeDtypeStruct((_B, 2 * _D), jnp.float32),
        scratch_types=[
            pltpu.VMEM((bw,), jnp.int32),
            pltpu.VMEM((bw, 2 * _D), jnp.float32),
            pltpu.SemaphoreType.DMA,
        ],
        compiler_params=pltpu.CompilerParams(needs_layout_passes=False),
    )
    def gather_k(table_hbm, idx_hbm, out_hbm, idx_v, rows_v, sem):
        wid = lax.axis_index("s") * info.num_cores + lax.axis_index("c")
        base = wid * bw
        pltpu.sync_copy(idx_hbm.at[pl.ds(base, bw)], idx_v)
        pltpu.async_copy(table_hbm.at[idx_v], rows_v, sem).wait()
        pltpu.sync_copy(rows_v, out_hbm.at[pl.ds(base, bw)])

    return gather_k(table_pad, target)


def _proj_t_body(wt_ref, embed_ref, b_ref, out_ref):
    acc = lax.dot_general(
        wt_ref[...],
        embed_ref[..., : _D],
        (((0,), (1,)), ((), ())),
        preferred_element_type=jnp.float32,
    )
    ones = jnp.ones((1, _B), jnp.float32)
    bias = lax.dot_general(
        b_ref[...][None, :],
        ones,
        (((0,), (0,)), ((), ())),
        preferred_element_type=jnp.float32,
    )
    out_ref[...] = acc + bias


def _tc_project_t(Wt, embed_pad, b):
    return pl.pallas_call(
        _proj_t_body,
        grid=(pl.cdiv(_V, _VT),),
        in_specs=[
            pl.BlockSpec((_D, _VT), lambda j: (0, j)),
            pl.BlockSpec((_B, 2 * _D), lambda j: (0, 0)),
            pl.BlockSpec((_VT,), lambda j: (j,)),
        ],
        out_specs=pl.BlockSpec((_VT, _B), lambda j: (j, 0)),
        out_shape=jax.ShapeDtypeStruct((_V, _B), jnp.float32),
    )(Wt, embed_pad, b)


def kernel(target, embeddings, W, b):
    pad_id = jnp.eye(_D, 2 * _D, dtype=jnp.float32)
    table_pad = lax.dot_general(
        embeddings, pad_id, (((1,), (0,)), ((), ())),
        preferred_element_type=jnp.float32,
    )
    embed_pad = _sc_gather(table_pad, target.astype(jnp.int32))
    out_t = _tc_project_t(W.T, embed_pad, b)
    return out_t.T


# speedup vs baseline: 3.9566x; 1.0120x over previous
import functools

import jax
import jax.numpy as jnp
from jax import lax
from jax.experimental import pallas as pl
from jax.experimental.pallas import tpu as pltpu
from jax.experimental.pallas import tpu_sc as plsc

_B = 1024
_D = 64
_V = 100000
_VT = 5120


def _sc_gather(table_pad, target):
    info = plsc.get_sparse_core_info()
    nw = info.num_cores * info.num_subcores
    bw = _B // nw

    mesh = plsc.VectorSubcoreMesh(core_axis_name="c", subcore_axis_name="s")

    @functools.partial(
        pl.kernel,
        mesh=mesh,
        out_type=jax.ShapeDtypeStruct((_B, 2 * _D), jnp.float32),
        scratch_types=[
            pltpu.VMEM((bw,), jnp.int32),
            pltpu.VMEM((bw, 2 * _D), jnp.float32),
            pltpu.SemaphoreType.DMA,
        ],
        compiler_params=pltpu.CompilerParams(needs_layout_passes=False),
    )
    def gather_k(table_hbm, idx_hbm, out_hbm, idx_v, rows_v, sem):
        wid = lax.axis_index("s") * info.num_cores + lax.axis_index("c")
        base = wid * bw
        pltpu.sync_copy(idx_hbm.at[pl.ds(base, bw)], idx_v)
        pltpu.async_copy(table_hbm.at[idx_v], rows_v, sem).wait()
        pltpu.sync_copy(rows_v, out_hbm.at[pl.ds(base, bw)])

    return gather_k(table_pad, target)


def _proj_t_body(wt_ref, embed_ref, b_ref, out_ref):
    acc = lax.dot_general(
        wt_ref[...],
        embed_ref[..., : _D],
        (((0,), (1,)), ((), ())),
        preferred_element_type=jnp.float32,
    )
    ones = jnp.ones((1, _B), jnp.float32)
    bias = lax.dot_general(
        b_ref[...][None, :],
        ones,
        (((0,), (0,)), ((), ())),
        preferred_element_type=jnp.float32,
    )
    out_ref[...] = acc + bias


def _tc_project_t(Wt, embed_pad, b):
    return pl.pallas_call(
        _proj_t_body,
        grid=(pl.cdiv(_V, _VT),),
        in_specs=[
            pl.BlockSpec((_D, _VT), lambda j: (0, j)),
            pl.BlockSpec((_B, 2 * _D), lambda j: (0, 0)),
            pl.BlockSpec((_VT,), lambda j: (j,)),
        ],
        out_specs=pl.BlockSpec((_VT, _B), lambda j: (j, 0)),
        out_shape=jax.ShapeDtypeStruct((_V, _B), jnp.float32),
    )(Wt, embed_pad, b)


def kernel(target, embeddings, W, b):
    pad_id = jnp.eye(_D, 2 * _D, dtype=jnp.float32)
    table_pad = lax.dot_general(
        embeddings, pad_id, (((1,), (0,)), ((), ())),
        preferred_element_type=jnp.float32,
    )
    embed_pad = _sc_gather(table_pad, target.astype(jnp.int32))
    out_t = _tc_project_t(W.T, embed_pad, b)
    return out_t.T
